# baseline (speedup 1.0000x reference)
import functools

import jax
import jax.numpy as jnp
from jax import lax
from jax.experimental import pallas as pl
from jax.experimental.pallas import tpu as pltpu
from jax.experimental.pallas import tpu_sc as plsc

N = 10000
NPAD = 10016
NB = 313
NBP = 320
NW = 32
NC = 2
E = 320000
CHUNK = 2560
NCHUNK = E // CHUNK
VREGS = CHUNK // 16
DRB = 2048
REGION = 158 * 2048
ACC_CAP = 4640
F = 160
SLAB = NBP * F
GK = 256

_mesh = plsc.VectorSubcoreMesh(core_axis_name="c", subcore_axis_name="s")
_sc_params = pltpu.CompilerParams(
    needs_layout_passes=False, use_tc_tiling_on_sc=False
)


@functools.partial(
    pl.kernel,
    out_type=[
        jax.ShapeDtypeStruct((NW * REGION,), jnp.int32),
        jax.ShapeDtypeStruct((NW * REGION,), jnp.int32),
        jax.ShapeDtypeStruct((NW * 128,), jnp.int32),
        jax.ShapeDtypeStruct((NW * 384,), jnp.float32),
    ],
    mesh=_mesh,
    compiler_params=_sc_params,
    scratch_types=[
        pltpu.VMEM((2, CHUNK), jnp.int32),
        pltpu.VMEM((2, CHUNK), jnp.int32),
        pltpu.VMEM((ACC_CAP,), jnp.int32),
        pltpu.VMEM((ACC_CAP,), jnp.int32),
        pltpu.VMEM((384,), jnp.float32),
        pltpu.VMEM((128,), jnp.int32),
        pltpu.SemaphoreType.DMA,
        pltpu.SemaphoreType.DMA,
    ],
)
def _sc_filter(ei, brow, bcol, counts, deg, rch, cch, racc, cacc, dacc, nvec, semA, semB):
    wid = lax.axis_index("s") * NC + lax.axis_index("c")
    lo = wid * NB
    hi = lo + NB
    base = wid * REGION
    zero16i = jnp.zeros((16,), jnp.int32)
    zero16f = jnp.zeros((16,), jnp.float32)
    ones16f = jnp.ones((16,), jnp.float32)

    trash16 = lax.broadcast(lo + NB, (16,))

    def z_body(i, _):
        racc[pl.ds(16 * i, 16)] = zero16i
        cacc[pl.ds(16 * i, 16)] = trash16
        return 0

    lax.fori_loop(0, ACC_CAP // 16, z_body, 0)
    for i in range(384 // 16):
        dacc[pl.ds(16 * i, 16)] = zero16f

    def drain(args):
        fill, off = args
        pltpu.sync_copy(racc.at[pl.ds(0, DRB)], brow.at[pl.ds(pl.multiple_of(base + off, DRB), DRB)])
        pltpu.sync_copy(cacc.at[pl.ds(0, DRB)], bcol.at[pl.ds(pl.multiple_of(base + off, DRB), DRB)])
        rem = fill - DRB

        def sh_body(j, _):
            racc[pl.ds(16 * j, 16)] = racc[pl.ds(DRB + 16 * j, 16)]
            cacc[pl.ds(16 * j, 16)] = cacc[pl.ds(DRB + 16 * j, 16)]
            return 0

        lax.fori_loop(0, (rem + 15) // 16, sh_body, 0)
        return rem, off + DRB

    csems = (semA, semB)

    def issue_chunk(c, b):
        cc = jnp.minimum(c, NCHUNK - 1)
        pltpu.async_copy(
            ei.at[pl.ds(pl.multiple_of(cc * CHUNK, 128), CHUNK)], rch.at[b], csems[b]
        )
        pltpu.async_copy(
            ei.at[pl.ds(pl.multiple_of(E + cc * CHUNK, 128), CHUNK)], cch.at[b], csems[b]
        )

    def wait_chunk(c, b):
        cc = jnp.minimum(c, NCHUNK - 1)
        pltpu.make_async_copy(
            ei.at[pl.ds(pl.multiple_of(cc * CHUNK, 128), CHUNK)], rch.at[b], csems[b]
        ).wait()
        pltpu.make_async_copy(
            ei.at[pl.ds(pl.multiple_of(E + cc * CHUNK, 128), CHUNK)], cch.at[b], csems[b]
        ).wait()

    def process_chunk(b, fill, off):
        def v_body(i, fill):
            for i4 in range(4):
                cv = cch[b, pl.ds(64 * i + 16 * i4, 16)]
                rv = rch[b, pl.ds(64 * i + 16 * i4, 16)]
                m = (cv >= lo) & (cv < hi)
                plsc.store_compressed(cacc.at[pl.ds(fill, 16)], cv, mask=m)
                plsc.store_compressed(racc.at[pl.ds(fill, 16)], rv, mask=m)
                plsc.addupdate_scatter(dacc, [cv - lo], ones16f, mask=m)
                fill = fill + plsc.all_reduce_population_count(m)[0]
            return fill

        fill = lax.fori_loop(0, VREGS // 4, v_body, fill)
        return lax.cond(fill >= DRB, drain, lambda a: a, (fill, off))

    issue_chunk(jnp.int32(0), 0)

    def chunk_pair_body(c2, carry):
        fill, off = carry
        for b in range(2):
            c = 2 * c2 + b
            issue_chunk(c + 1, 1 - b)
            wait_chunk(c, b)
            fill, off = process_chunk(b, fill, off)
        return fill, off

    fill, off = lax.fori_loop(
        0, NCHUNK // 2, chunk_pair_body, (jnp.int32(0), jnp.int32(0))
    )
    wait_chunk(jnp.int32(NCHUNK - 1), 0)
    fill, off = process_chunk(0, fill, off)
    pltpu.sync_copy(racc.at[pl.ds(0, DRB)], brow.at[pl.ds(pl.multiple_of(base + off, DRB), DRB)])
    pltpu.sync_copy(cacc.at[pl.ds(0, DRB)], bcol.at[pl.ds(pl.multiple_of(base + off, DRB), DRB)])
    for i in range(128 // 16):
        nvec[pl.ds(16 * i, 16)] = lax.broadcast(off + fill, (16,))
    pltpu.sync_copy(nvec, counts.at[pl.ds(pl.multiple_of(128 * wid, 128), 128)])
    pltpu.sync_copy(dacc, deg.at[pl.ds(pl.multiple_of(384 * wid, 128), 384)])


@functools.partial(
    pl.kernel,
    out_type=jax.ShapeDtypeStruct((NPAD, F), jnp.bfloat16),
    mesh=_mesh,
    compiler_params=_sc_params,
    scratch_types=[
        pltpu.VMEM((NBP, F), jnp.bfloat16),
        pltpu.VMEM((NBP, F), jnp.bfloat16),
        pltpu.VMEM((2, GK, F), jnp.bfloat16),
        pltpu.VMEM((DRB + GK,), jnp.int32),
        pltpu.VMEM((DRB + 16,), jnp.int32),
        pltpu.VMEM((2, GK), jnp.int32),
        pltpu.VMEM((128,), jnp.int32),
        pltpu.SemaphoreType.DMA,
        pltpu.SemaphoreType.DMA,
    ],
)
def _sc_segmax(g2d, brow, bcol, counts, out, acc, acc1, grows, rbuf, cbuf, gidx, cntv, sem0, sem1):
    wid = lax.axis_index("s") * NC + lax.axis_index("c")
    lo = wid * NB
    base = wid * REGION
    pltpu.sync_copy(counts.at[pl.ds(pl.multiple_of(128 * wid, 128), 128)], cntv)
    n = cntv[pl.ds(0, 16)][0]
    pltpu.sync_copy(g2d.at[pl.ds(lo, NB)], acc.at[pl.ds(0, NB)])
    pltpu.sync_copy(g2d.at[pl.ds(lo, NB)], acc1.at[pl.ds(0, NB)])
    iota16 = lax.iota(jnp.int32, 16)
    for u in range(GK // 16):
        rbuf[pl.ds(DRB + 16 * u, 16)] = iota16 + (16 * u)

    sems = (sem0, sem1)

    def issue(s, b):
        for u in range(GK // 16):
            gidx[b, pl.ds(16 * u, 16)] = rbuf[pl.ds(GK * s + 16 * u, 16)]
        pltpu.async_copy(g2d.at[gidx.at[b]], grows.at[b], sems[b])

    def wait(b):
        pltpu.make_async_copy(g2d.at[gidx.at[b]], grows.at[b], sems[b]).wait()

    def rmw(s, b):
        def rmw8(jj, _):
            eb = GK * s + 8 * jj
            for j8 in range(8):
                bank = acc if j8 % 2 == 0 else acc1
                lc = cbuf[pl.ds(eb + j8, 16)][0] - lo
                row = 8 * jj + j8
                vals = [
                    jnp.maximum(
                        bank[lc, pl.ds(32 * k, 32)],
                        grows[b, row, pl.ds(32 * k, 32)],
                    )
                    for k in range(F // 32)
                ]
                for k in range(F // 32):
                    bank[lc, pl.ds(32 * k, 32)] = vals[k]
            return 0

        lax.fori_loop(0, GK // 8, rmw8, 0)

    def blk_body(t, _):
        pltpu.sync_copy(brow.at[pl.ds(pl.multiple_of(base + DRB * t, DRB), DRB)], rbuf.at[pl.ds(0, DRB)])
        pltpu.sync_copy(bcol.at[pl.ds(pl.multiple_of(base + DRB * t, DRB), DRB)], cbuf.at[pl.ds(0, DRB)])
        issue(jnp.int32(0), 0)

        def pair_body(s2, _):
            s = 2 * s2
            issue(s + 1, 1)
            wait(0)
            rmw(s, 0)
            issue(s + 2, 0)
            wait(1)
            rmw(s + 1, 1)
            return 0

        lax.fori_loop(0, DRB // GK // 2, pair_body, 0)
        wait(0)
        return 0

    lax.fori_loop(0, (n + DRB - 1) // DRB, blk_body, 0)

    def merge_body(r, _):
        for k in range(F // 32):
            acc[r, pl.ds(32 * k, 32)] = jnp.maximum(
                acc[r, pl.ds(32 * k, 32)], acc1[r, pl.ds(32 * k, 32)]
            )
        return 0

    lax.fori_loop(0, NB, merge_body, 0)
    pltpu.sync_copy(acc.at[pl.ds(0, NB)], out.at[pl.ds(lo, NB)])


def _t1(xp, W1p, degv):
    def body(x_ref, w_ref, d_ref, o1_ref, o2_ref):
        dis = lax.rsqrt(d_ref[...] + 1.0)
        h = jnp.dot(x_ref[...], w_ref[...], preferred_element_type=jnp.float32)
        gfull = (h * dis).astype(jnp.bfloat16)
        o1_ref[...] = gfull[:, :F]
        o2_ref[...] = gfull[:, F:]

    return pl.pallas_call(
        body,
        out_shape=[jax.ShapeDtypeStruct((NPAD, F), jnp.bfloat16)] * 2,
    )(xp, W1p, degv)


def _t2(m1a, m1b, degv, b1, W2):
    def body(a_ref, b_ref, d_ref, bias_ref, w_ref, o_ref):
        dis = lax.rsqrt(d_ref[...] + 1.0)
        m1 = jnp.concatenate([a_ref[...], b_ref[...]], axis=1).astype(jnp.float32)
        h = m1 * dis + bias_ref[...]
        h = jnp.maximum(h, 0.0)
        g2 = jnp.dot(h, w_ref[...], preferred_element_type=jnp.float32) * dis
        o_ref[...] = g2.astype(jnp.bfloat16)

    return pl.pallas_call(
        body,
        out_shape=jax.ShapeDtypeStruct((NPAD, F), jnp.bfloat16),
    )(m1a, m1b, degv, b1, W2)


def _t3(m2, degv, b2, W3p, b3p):
    def body(m_ref, d_ref, bias_ref, w_ref, b3_ref, o_ref):
        dis = lax.rsqrt(d_ref[...] + 1.0)
        h = m_ref[...].astype(jnp.float32) * dis + bias_ref[...]
        o_ref[...] = jnp.dot(h, w_ref[...], preferred_element_type=jnp.float32) + b3_ref[...]

    return pl.pallas_call(
        body,
        out_shape=jax.ShapeDtypeStruct((NPAD, 128), jnp.float32),
    )(m2, degv, b2, W3p, b3p)


def kernel(x, edge_index, W1, b1, W2, b2, W3, b3):
    ei_flat = edge_index.reshape(2 * E)
    brow, bcol, counts, deg = _sc_filter(ei_flat)
    degv = deg.reshape(NW, 384)[:, :NB].reshape(NPAD, 1)
    xp = jnp.pad(x, ((0, NPAD - N), (0, 3)))
    W1p = jnp.pad(W1, ((0, 3), (0, 0)))
    g1a, g1b = _t1(xp, W1p, degv)
    m1a = _sc_segmax(g1a, brow, bcol, counts)
    m1b = _sc_segmax(g1b, brow, bcol, counts)
    g2 = _t2(m1a, m1b, degv, b1, W2)
    m2 = _sc_segmax(g2, brow, bcol, counts)
    W3p = jnp.pad(W3, ((0, 0), (0, 126)))
    b3p = jnp.pad(b3, (0, 126))
    out = _t3(m2, degv, b2, W3p, b3p)
    return out[:N, :2]


# speedup vs baseline: 1.4739x; 1.4739x over previous
import functools

import jax
import jax.numpy as jnp
from jax import lax
from jax.experimental import pallas as pl
from jax.experimental.pallas import tpu as pltpu
from jax.experimental.pallas import tpu_sc as plsc

N = 10000
NPAD = 10016
NB = 313
NBP = 320
NW = 32
NC = 2
E = 320000
CHUNK = 2560
NCHUNK = E // CHUNK
VREGS = CHUNK // 16
DRB = 2048
REGION = 158 * 2048
ACC_CAP = 4640
F = 160
SLAB = NBP * F
GK = 256

_mesh = plsc.VectorSubcoreMesh(core_axis_name="c", subcore_axis_name="s")
_sc_params = pltpu.CompilerParams(
    needs_layout_passes=False, use_tc_tiling_on_sc=False
)


@functools.partial(
    pl.kernel,
    out_type=[
        jax.ShapeDtypeStruct((NW * REGION,), jnp.int32),
        jax.ShapeDtypeStruct((NW * REGION,), jnp.int32),
        jax.ShapeDtypeStruct((NW * 128,), jnp.int32),
        jax.ShapeDtypeStruct((NW * 384,), jnp.float32),
    ],
    mesh=_mesh,
    compiler_params=_sc_params,
    scratch_types=[
        pltpu.VMEM((2, CHUNK), jnp.int32),
        pltpu.VMEM((2, CHUNK), jnp.int32),
        pltpu.VMEM((ACC_CAP,), jnp.int32),
        pltpu.VMEM((ACC_CAP,), jnp.int32),
        pltpu.VMEM((384,), jnp.float32),
        pltpu.VMEM((128,), jnp.int32),
        pltpu.SemaphoreType.DMA,
        pltpu.SemaphoreType.DMA,
    ],
)
def _sc_filter(ei, brow, bcol, counts, deg, rch, cch, racc, cacc, dacc, nvec, semA, semB):
    wid = lax.axis_index("s") * NC + lax.axis_index("c")
    lo = wid * NB
    hi = lo + NB
    base = wid * REGION
    zero16i = jnp.zeros((16,), jnp.int32)
    zero16f = jnp.zeros((16,), jnp.float32)
    ones16f = jnp.ones((16,), jnp.float32)

    trash16 = lax.broadcast(lo + NB, (16,))

    def z_body(i, _):
        racc[pl.ds(16 * i, 16)] = zero16i
        cacc[pl.ds(16 * i, 16)] = trash16
        return 0

    lax.fori_loop(0, ACC_CAP // 16, z_body, 0)
    for i in range(384 // 16):
        dacc[pl.ds(16 * i, 16)] = zero16f

    def drain(args):
        fill, off = args
        pltpu.sync_copy(racc.at[pl.ds(0, DRB)], brow.at[pl.ds(pl.multiple_of(base + off, DRB), DRB)])
        pltpu.sync_copy(cacc.at[pl.ds(0, DRB)], bcol.at[pl.ds(pl.multiple_of(base + off, DRB), DRB)])
        rem = fill - DRB

        def sh_body(j, _):
            racc[pl.ds(16 * j, 16)] = racc[pl.ds(DRB + 16 * j, 16)]
            cacc[pl.ds(16 * j, 16)] = cacc[pl.ds(DRB + 16 * j, 16)]
            return 0

        lax.fori_loop(0, (rem + 15) // 16, sh_body, 0)
        return rem, off + DRB

    csems = (semA, semB)

    def issue_chunk(c, b):
        cc = jnp.minimum(c, NCHUNK - 1)
        pltpu.async_copy(
            ei.at[pl.ds(pl.multiple_of(cc * CHUNK, 128), CHUNK)], rch.at[b], csems[b]
        )
        pltpu.async_copy(
            ei.at[pl.ds(pl.multiple_of(E + cc * CHUNK, 128), CHUNK)], cch.at[b], csems[b]
        )

    def wait_chunk(c, b):
        cc = jnp.minimum(c, NCHUNK - 1)
        pltpu.make_async_copy(
            ei.at[pl.ds(pl.multiple_of(cc * CHUNK, 128), CHUNK)], rch.at[b], csems[b]
        ).wait()
        pltpu.make_async_copy(
            ei.at[pl.ds(pl.multiple_of(E + cc * CHUNK, 128), CHUNK)], cch.at[b], csems[b]
        ).wait()

    def process_chunk(b, fill, off):
        def v_body(i, fill):
            for i4 in range(4):
                cv = cch[b, pl.ds(64 * i + 16 * i4, 16)]
                rv = rch[b, pl.ds(64 * i + 16 * i4, 16)]
                m = (cv >= lo) & (cv < hi)
                plsc.store_compressed(cacc.at[pl.ds(fill, 16)], cv, mask=m)
                plsc.store_compressed(racc.at[pl.ds(fill, 16)], rv, mask=m)
                plsc.addupdate_scatter(dacc, [cv - lo], ones16f, mask=m)
                fill = fill + plsc.all_reduce_population_count(m)[0]
            return fill

        fill = lax.fori_loop(0, VREGS // 4, v_body, fill)
        return lax.cond(fill >= DRB, drain, lambda a: a, (fill, off))

    issue_chunk(jnp.int32(0), 0)

    def chunk_pair_body(c2, carry):
        fill, off = carry
        for b in range(2):
            c = 2 * c2 + b
            issue_chunk(c + 1, 1 - b)
            wait_chunk(c, b)
            fill, off = process_chunk(b, fill, off)
        return fill, off

    fill, off = lax.fori_loop(
        0, NCHUNK // 2, chunk_pair_body, (jnp.int32(0), jnp.int32(0))
    )
    wait_chunk(jnp.int32(NCHUNK - 1), 0)
    fill, off = process_chunk(0, fill, off)
    pltpu.sync_copy(racc.at[pl.ds(0, DRB)], brow.at[pl.ds(pl.multiple_of(base + off, DRB), DRB)])
    pltpu.sync_copy(cacc.at[pl.ds(0, DRB)], bcol.at[pl.ds(pl.multiple_of(base + off, DRB), DRB)])
    for i in range(128 // 16):
        nvec[pl.ds(16 * i, 16)] = lax.broadcast(off + fill, (16,))
    pltpu.sync_copy(nvec, counts.at[pl.ds(pl.multiple_of(128 * wid, 128), 128)])
    pltpu.sync_copy(dacc, deg.at[pl.ds(pl.multiple_of(384 * wid, 128), 384)])


@functools.partial(
    pl.kernel,
    out_type=jax.ShapeDtypeStruct((NPAD, F), jnp.bfloat16),
    mesh=_mesh,
    compiler_params=_sc_params,
    scratch_types=[
        pltpu.VMEM((NBP, F), jnp.bfloat16),
        pltpu.VMEM((2, GK, F), jnp.bfloat16),
        pltpu.VMEM((DRB + GK,), jnp.int32),
        pltpu.VMEM((DRB + 16,), jnp.int32),
        pltpu.VMEM((2, GK), jnp.int32),
        pltpu.VMEM((128,), jnp.int32),
        pltpu.SemaphoreType.DMA,
        pltpu.SemaphoreType.DMA,
    ],
)
def _sc_segmax(g2d, brow, bcol, counts, out, acc, grows, rbuf, cbuf, gidx, cntv, sem0, sem1):
    wid = lax.axis_index("s") * NC + lax.axis_index("c")
    lo = wid * NB
    base = wid * REGION
    pltpu.sync_copy(counts.at[pl.ds(pl.multiple_of(128 * wid, 128), 128)], cntv)
    n = cntv[pl.ds(0, 16)][0]
    pltpu.sync_copy(g2d.at[pl.ds(lo, NB)], acc.at[pl.ds(0, NB)])
    iota16 = lax.iota(jnp.int32, 16)
    for u in range(GK // 16):
        rbuf[pl.ds(DRB + 16 * u, 16)] = iota16 + (16 * u)

    sems = (sem0, sem1)

    def issue(s, b):
        for u in range(GK // 16):
            gidx[b, pl.ds(16 * u, 16)] = rbuf[pl.ds(GK * s + 16 * u, 16)]
        pltpu.async_copy(g2d.at[gidx.at[b]], grows.at[b], sems[b])

    def wait(b):
        pltpu.make_async_copy(g2d.at[gidx.at[b]], grows.at[b], sems[b]).wait()

    def rmw(s, b):
        def rmw16(jj, _):
            eb = GK * s + 16 * jj
            cvec = cbuf[pl.ds(eb, 16)] - lo
            for j in range(16):
                lc = cvec[j]
                row = 16 * jj + j
                vals = [
                    jnp.maximum(
                        acc[lc, pl.ds(32 * k, 32)],
                        grows[b, row, pl.ds(32 * k, 32)],
                    )
                    for k in range(F // 32)
                ]
                for k in range(F // 32):
                    acc[lc, pl.ds(32 * k, 32)] = vals[k]
            return 0

        lax.fori_loop(0, GK // 16, rmw16, 0)

    def blk_body(t, _):
        pltpu.sync_copy(brow.at[pl.ds(pl.multiple_of(base + DRB * t, DRB), DRB)], rbuf.at[pl.ds(0, DRB)])
        pltpu.sync_copy(bcol.at[pl.ds(pl.multiple_of(base + DRB * t, DRB), DRB)], cbuf.at[pl.ds(0, DRB)])
        issue(jnp.int32(0), 0)

        def pair_body(s2, _):
            s = 2 * s2
            issue(s + 1, 1)
            wait(0)
            rmw(s, 0)
            issue(s + 2, 0)
            wait(1)
            rmw(s + 1, 1)
            return 0

        lax.fori_loop(0, DRB // GK // 2, pair_body, 0)
        wait(0)
        return 0

    lax.fori_loop(0, (n + DRB - 1) // DRB, blk_body, 0)
    pltpu.sync_copy(acc.at[pl.ds(0, NB)], out.at[pl.ds(lo, NB)])


def _t1(xp, W1p, degv):
    def body(x_ref, w_ref, d_ref, o1_ref, o2_ref):
        dis = lax.rsqrt(d_ref[...] + 1.0)
        h = jnp.dot(x_ref[...], w_ref[...], preferred_element_type=jnp.float32)
        gfull = (h * dis).astype(jnp.bfloat16)
        o1_ref[...] = gfull[:, :F]
        o2_ref[...] = gfull[:, F:]

    return pl.pallas_call(
        body,
        out_shape=[jax.ShapeDtypeStruct((NPAD, F), jnp.bfloat16)] * 2,
    )(xp, W1p, degv)


def _t2(m1a, m1b, degv, b1, W2):
    def body(a_ref, b_ref, d_ref, bias_ref, w_ref, o_ref):
        dis = lax.rsqrt(d_ref[...] + 1.0)
        m1 = jnp.concatenate([a_ref[...], b_ref[...]], axis=1).astype(jnp.float32)
        h = m1 * dis + bias_ref[...]
        h = jnp.maximum(h, 0.0)
        g2 = jnp.dot(h, w_ref[...], preferred_element_type=jnp.float32) * dis
        o_ref[...] = g2.astype(jnp.bfloat16)

    return pl.pallas_call(
        body,
        out_shape=jax.ShapeDtypeStruct((NPAD, F), jnp.bfloat16),
    )(m1a, m1b, degv, b1, W2)


def _t3(m2, degv, b2, W3p, b3p):
    def body(m_ref, d_ref, bias_ref, w_ref, b3_ref, o_ref):
        dis = lax.rsqrt(d_ref[...] + 1.0)
        h = m_ref[...].astype(jnp.float32) * dis + bias_ref[...]
        o_ref[...] = jnp.dot(h, w_ref[...], preferred_element_type=jnp.float32) + b3_ref[...]

    return pl.pallas_call(
        body,
        out_shape=jax.ShapeDtypeStruct((NPAD, 128), jnp.float32),
    )(m2, degv, b2, W3p, b3p)


def kernel(x, edge_index, W1, b1, W2, b2, W3, b3):
    ei_flat = edge_index.reshape(2 * E)
    brow, bcol, counts, deg = _sc_filter(ei_flat)
    degv = deg.reshape(NW, 384)[:, :NB].reshape(NPAD, 1)
    xp = jnp.pad(x, ((0, NPAD - N), (0, 3)))
    W1p = jnp.pad(W1, ((0, 3), (0, 0)))
    g1a, g1b = _t1(xp, W1p, degv)
    m1a = _sc_segmax(g1a, brow, bcol, counts)
    m1b = _sc_segmax(g1b, brow, bcol, counts)
    g2 = _t2(m1a, m1b, degv, b1, W2)
    m2 = _sc_segmax(g2, brow, bcol, counts)
    W3p = jnp.pad(W3, ((0, 0), (0, 126)))
    b3p = jnp.pad(b3, (0, 126))
    out = _t3(m2, degv, b2, W3p, b3p)
    return out[:N, :2]
